# speedup vs baseline: 9.2618x; 9.2618x over previous
import functools

import jax
import jax.numpy as jnp
from jax import lax
from jax.experimental import pallas as pl
from jax.experimental.pallas import tpu as pltpu
from jax.experimental.pallas import tpu_sc as plsc

B = 4096
L = 200
D = 128
N = B * L
NC = 2
NS = 16
NW = NC * NS
PER_W = N // NW
CHUNK = 128
NCHUNK = PER_W // CHUNK

_mesh = plsc.VectorSubcoreMesh(core_axis_name="c", subcore_axis_name="s")


@functools.partial(
    pl.kernel,
    mesh=_mesh,
    out_type=jax.ShapeDtypeStruct((N, D), jnp.float32),
    scratch_types=[
        pltpu.VMEM((NCHUNK, CHUNK), jnp.int32),
        pltpu.VMEM((CHUNK, D), jnp.float32),
        pltpu.VMEM((CHUNK, D), jnp.float32),
        pltpu.SemaphoreType.DMA,
    ],
)
def _gather_kernel(x_hbm, w_hbm, out_hbm, idx_v, buf0, buf1, gsem):
    wid = lax.axis_index("s") * NC + lax.axis_index("c")
    base = wid * PER_W
    pltpu.sync_copy(x_hbm.at[wid], idx_v)

    bufs = (buf0, buf1)
    pltpu.async_copy(w_hbm.at[idx_v.at[0]], buf0, gsem)
    pltpu.async_copy(w_hbm.at[idx_v.at[1]], buf1, gsem)

    def body(g, carry):
        for b in range(2):
            j = g * 2 + b
            buf = bufs[b]
            pltpu.make_async_copy(w_hbm.at[idx_v.at[0]], buf, gsem).wait()
            pltpu.sync_copy(buf, out_hbm.at[pl.ds(base + j * CHUNK, CHUNK)])

            @pl.when(j + 2 < NCHUNK)
            def _():
                pltpu.async_copy(w_hbm.at[idx_v.at[j + 2]], buf, gsem)

        return carry

    lax.fori_loop(0, NCHUNK // 2, body, 0)


def kernel(x, target, text_inputs, W):
    del target, text_inputs
    x3 = x.reshape(NW, NCHUNK, CHUNK)
    out = _gather_kernel(x3, W)
    return out.reshape(B, L, D)
